# Initial kernel scaffold; baseline (speedup 1.0000x reference)
import jax
import jax.numpy as jnp
from jax.experimental import pallas as pl


def kernel(x, edge_idx, edge_attr, u, W1, b1, W2, b2, g1, be1, W3, b3, W4, b4, g2, be2):
    raise NotImplementedError("write your pallas kernel here")


# speedup vs baseline: 5.7123x; 5.7123x over previous
import functools

import jax
import jax.numpy as jnp
from jax import lax
from jax.experimental import pallas as pl
from jax.experimental.pallas import tpu as pltpu
from jax.experimental.pallas import tpu_sc as plsc

N_NODES = 10000
N_EDGES = 320000
N_FEAT = 128
N_HID = 16
PACK = 128 // N_HID
EP = N_EDGES // PACK
NP = N_NODES // PACK
EPS = 1e-5

NC, NS = 2, 16
NW = NC * NS
E_PER_W = N_EDGES // NW
CHUNK = 2000
N_CHUNKS = E_PER_W // CHUNK
NPAD = 10240
STRIPE = NPAD // NS

_HI = jax.lax.Precision.HIGHEST


def _xw_body(x_ref, w_ref, o_ref):
    o_ref[...] = jnp.dot(x_ref[...], w_ref[...], precision=_HI)


def _tc_xw(x, w):
    return pl.pallas_call(
        _xw_body,
        out_shape=jax.ShapeDtypeStruct((x.shape[0], w.shape[1]), jnp.float32),
    )(x, w)


def _sc_gather_build():
    mesh = plsc.VectorSubcoreMesh(core_axis_name="c", subcore_axis_name="s")

    @functools.partial(
        pl.kernel,
        mesh=mesh,
        out_type=jax.ShapeDtypeStruct((N_EDGES, N_HID), jnp.float32),
        scratch_types=[
            pltpu.VMEM((CHUNK,), jnp.int32),
            pltpu.VMEM((CHUNK, N_HID), jnp.float32),
            pltpu.SemaphoreType.DMA,
        ],
        compiler_params=pltpu.CompilerParams(use_tc_tiling_on_sc=False),
    )
    def gather_k(table_hbm, idx_hbm, out_hbm, idx_v, rows_v, sem):
        wid = lax.axis_index("s") * NC + lax.axis_index("c")
        for j in range(N_CHUNKS):
            base = wid * E_PER_W + j * CHUNK
            pltpu.sync_copy(idx_hbm.at[pl.ds(base, CHUNK)], idx_v)
            pltpu.async_copy(table_hbm.at[idx_v], rows_v, sem).wait()
            pltpu.sync_copy(rows_v, out_hbm.at[pl.ds(base, CHUNK)])

    return gather_k


_gather_cache = []


def _sc_gather(table, idx):
    if not _gather_cache:
        _gather_cache.append(_sc_gather_build())
    return _gather_cache[0](table, idx)


def _sc_scatter_build():
    mesh = plsc.VectorSubcoreMesh(core_axis_name="c", subcore_axis_name="s")

    @functools.partial(
        pl.kernel,
        mesh=mesh,
        out_type=(
            jax.ShapeDtypeStruct((NC, NPAD, N_HID), jnp.float32),
            jax.ShapeDtypeStruct((NC, NPAD), jnp.float32),
        ),
        scratch_types=[
            pltpu.VMEM_SHARED((NPAD, N_HID), jnp.float32),
            pltpu.VMEM_SHARED((NPAD,), jnp.float32),
            pltpu.VMEM((CHUNK,), jnp.int32),
            pltpu.VMEM((CHUNK, N_HID), jnp.float32),
            pltpu.VMEM((CHUNK,), jnp.float32),
            pltpu.VMEM((STRIPE, N_HID), jnp.float32),
            pltpu.VMEM((STRIPE,), jnp.float32),
        ],
        compiler_params=pltpu.CompilerParams(use_tc_tiling_on_sc=False),
    )
    def scatter_k(row_hbm, h_hbm, acc_out, cnt_out,
                  acc_s, cnt_s, idx_v, h_v, ones_v, zrow_v, zcnt_v):
        c = lax.axis_index("c")
        s = lax.axis_index("s")

        def fill_rows(i, _):
            zrow_v[i] = jnp.zeros((N_HID,), jnp.float32)
            return 0

        lax.fori_loop(0, STRIPE, fill_rows, 0)

        def fill_1d(i, _):
            zcnt_v[pl.ds(i * 16, 16)] = jnp.zeros((16,), jnp.float32)
            ones_v[pl.ds(i * 16, 16)] = jnp.ones((16,), jnp.float32)
            return 0

        lax.fori_loop(0, STRIPE // 16, fill_1d, 0)

        def fill_ones_tail(i, _):
            ones_v[pl.ds(i * 16, 16)] = jnp.ones((16,), jnp.float32)
            return 0

        lax.fori_loop(STRIPE // 16, CHUNK // 16, fill_ones_tail, 0)

        pltpu.sync_copy(zrow_v, acc_s.at[pl.ds(s * STRIPE, STRIPE)])
        pltpu.sync_copy(zcnt_v, cnt_s.at[pl.ds(s * STRIPE, STRIPE)])
        plsc.subcore_barrier()

        for j in range(N_CHUNKS):
            base = c * (N_EDGES // NC) + s * E_PER_W + j * CHUNK
            pltpu.sync_copy(row_hbm.at[pl.ds(base, CHUNK)], idx_v)
            pltpu.sync_copy(h_hbm.at[pl.ds(base, CHUNK)], h_v)
            pltpu.sync_copy(h_v, acc_s.at[idx_v], add=True)
            pltpu.sync_copy(ones_v, cnt_s.at[idx_v], add=True)

        plsc.subcore_barrier()
        pltpu.sync_copy(acc_s.at[pl.ds(s * STRIPE, STRIPE)],
                        acc_out.at[c, pl.ds(s * STRIPE, STRIPE)])
        pltpu.sync_copy(cnt_s.at[pl.ds(s * STRIPE, STRIPE)],
                        cnt_out.at[c, pl.ds(s * STRIPE, STRIPE)])

    return scatter_k


_scatter_cache = []


def _sc_scatter(row, h):
    if not _scatter_cache:
        _scatter_cache.append(_sc_scatter_build())
    return _scatter_cache[0](row, h)


def _edge_body(g_ref, ea_ref, w1e_ref, w2c_ref, s16_ref,
               b1_ref, b2c_ref, g1_ref, be1_ref, o_ref):
    t = jnp.dot(ea_ref[...], w1e_ref[...], precision=_HI)
    p = g_ref[...] + t + b1_ref[...]
    r = jnp.maximum(p, 0.0)
    cc = jnp.dot(r, w2c_ref[...], precision=_HI) + b2c_ref[...]
    v = jnp.dot(cc * cc, s16_ref[...], precision=_HI)
    o_ref[...] = cc * jax.lax.rsqrt(v + EPS) * g1_ref[...] + be1_ref[...]


def _tc_edge(g_p, ea_p, w1e_blk, w2c_blk, s16, b1_t, b2c_t, g1_t, be1_t):
    grid = 8
    eb = EP // grid
    row_spec = pl.BlockSpec((eb, 128), lambda i: (i, 0))
    full = pl.BlockSpec((128, 128), lambda i: (0, 0))
    vec = pl.BlockSpec((1, 128), lambda i: (0, 0))
    return pl.pallas_call(
        _edge_body,
        grid=(grid,),
        in_specs=[row_spec, row_spec, full, full, full, vec, vec, vec, vec],
        out_specs=row_spec,
        out_shape=jax.ShapeDtypeStruct((EP, 128), jnp.float32),
    )(g_p, ea_p, w1e_blk, w2c_blk, s16, b1_t, b2c_t, g1_t, be1_t)


def _node_body(xr_ref, acc_ref, cnt_ref, u_ref, w3x_ref, w3m_ref, w3u_ref,
               b8_ref, w4c_ref, s16_ref, b3_ref, b4c_ref, g2_ref, be2_ref,
               o_ref):
    acc = acc_ref[0] + acc_ref[1]
    cntn = cnt_ref[0] + cnt_ref[1]
    cnt_p = jnp.dot(cntn, b8_ref[...], precision=_HI)
    mean_p = acc / jnp.maximum(cnt_p, 1.0)
    z = (jnp.dot(xr_ref[...], w3x_ref[...], precision=_HI)
         + jnp.dot(mean_p, w3m_ref[...], precision=_HI)
         + jnp.dot(u_ref[...], w3u_ref[...], precision=_HI)
         + b3_ref[...])
    r = jnp.maximum(z, 0.0)
    cc = jnp.dot(r, w4c_ref[...], precision=_HI) + b4c_ref[...]
    v = jnp.dot(cc * cc, s16_ref[...], precision=_HI)
    o_ref[...] = cc * jax.lax.rsqrt(v + EPS) * g2_ref[...] + be2_ref[...]


def _tc_node(xr, acc_p, cnt_r, u_p, w3x_big, w3m_blk, w3u_blk, b8, w4c_blk,
             s16, b3_t, b4c_t, g2_t, be2_t):
    return pl.pallas_call(
        _node_body,
        out_shape=jax.ShapeDtypeStruct((NP, 128), jnp.float32),
    )(xr, acc_p, cnt_r, u_p, w3x_big, w3m_blk, w3u_blk, b8, w4c_blk,
      s16, b3_t, b4c_t, g2_t, be2_t)


def kernel(x, edge_idx, edge_attr, u, W1, b1, W2, b2, g1, be1,
           W3, b3, W4, b4, g2, be2):
    row = edge_idx[0].astype(jnp.int32)
    col = edge_idx[1].astype(jnp.int32)

    i8 = jnp.eye(PACK, dtype=jnp.float32)
    W1x = W1[:N_FEAT]
    w1e_blk = jnp.kron(i8, W1[N_FEAT:])
    w2c = W2 - jnp.mean(W2, axis=1, keepdims=True)
    w2c_blk = jnp.kron(i8, w2c)
    w4c = W4 - jnp.mean(W4, axis=1, keepdims=True)
    w4c_blk = jnp.kron(i8, w4c)
    w3x_big = jnp.kron(i8, W3[:N_FEAT])
    w3m_blk = jnp.kron(i8, W3[N_FEAT:N_FEAT + N_HID])
    w3u_blk = jnp.kron(i8, W3[N_FEAT + N_HID:])
    s16 = jnp.kron(i8, jnp.full((N_HID, N_HID), 1.0 / N_HID, jnp.float32))
    b8 = jnp.kron(i8, jnp.ones((1, N_HID), jnp.float32))

    def tile8(v):
        return jnp.tile(v, PACK).reshape(1, 128)

    b1_t = tile8(b1)
    b2c_t = tile8(b2 - jnp.mean(b2))
    b4c_t = tile8(b4 - jnp.mean(b4))
    b3_t = tile8(b3)
    g1_t, be1_t = tile8(g1), tile8(be1)
    g2_t, be2_t = tile8(g2), tile8(be2)

    xw1 = _tc_xw(x, W1x)

    g = _sc_gather(xw1, col)

    g_p = g.reshape(EP, 128)
    ea_p = edge_attr.reshape(EP, 128)
    h_p = _tc_edge(g_p, ea_p, w1e_blk, w2c_blk, s16, b1_t, b2c_t, g1_t, be1_t)
    h = h_p.reshape(N_EDGES, N_HID)

    acc2, cnt2 = _sc_scatter(row, h)

    acc_p = acc2[:, :N_NODES].reshape(NC, NP, 128)
    cnt_r = cnt2[:, :N_NODES].reshape(NC, NP, PACK)
    xr = x.reshape(NP, PACK * N_FEAT)
    u_p = u.reshape(NP, 128)
    out_p = _tc_node(xr, acc_p, cnt_r, u_p, w3x_big, w3m_blk, w3u_blk, b8,
                     w4c_blk, s16, b3_t, b4c_t, g2_t, be2_t)
    return out_p.reshape(N_NODES, N_HID)


# speedup vs baseline: 8.8367x; 1.5470x over previous
import functools

import jax
import jax.numpy as jnp
from jax import lax
from jax.experimental import pallas as pl
from jax.experimental.pallas import tpu as pltpu
from jax.experimental.pallas import tpu_sc as plsc

N_NODES = 10000
N_EDGES = 320000
N_FEAT = 128
N_HID = 16
PACK = 128 // N_HID
EP = N_EDGES // PACK
NP = N_NODES // PACK
EPS = 1e-5

NC, NS = 2, 16
NW = NC * NS
E_PER_W = N_EDGES // NW
CHUNK = 2000
N_CHUNKS = E_PER_W // CHUNK
NPAD = 10240
NPP = NPAD // PACK
STRIPE = NPAD // NS


def _xw_body(xr_ref, w_ref, o_ref):
    o_ref[...] = jnp.dot(xr_ref[...], w_ref[...])


def _tc_xw(xr, wbig):
    return pl.pallas_call(
        _xw_body,
        out_shape=jax.ShapeDtypeStruct((NP, 128), jnp.float32),
    )(xr, wbig)


def _tt_body(w_ref, ea_ref, b_ref, o_ref):
    o_ref[...] = jnp.dot(w_ref[...], ea_ref[...]) + b_ref[...]


def _tc_tt(w1t, eaT, b1c):
    grid = 10
    eb = N_EDGES // grid
    col_spec = pl.BlockSpec((N_HID, eb), lambda i: (0, i))
    return pl.pallas_call(
        _tt_body,
        grid=(grid,),
        in_specs=[pl.BlockSpec((N_HID, N_HID), lambda i: (0, 0)),
                  col_spec,
                  pl.BlockSpec((N_HID, 1), lambda i: (0, 0))],
        out_specs=col_spec,
        out_shape=jax.ShapeDtypeStruct((N_HID, N_EDGES), jnp.float32),
    )(w1t, eaT, b1c)


def _sc_gather_build():
    mesh = plsc.VectorSubcoreMesh(core_axis_name="c", subcore_axis_name="s")

    @functools.partial(
        pl.kernel,
        mesh=mesh,
        out_type=jax.ShapeDtypeStruct((N_EDGES, N_HID), jnp.float32),
        scratch_types=[
            pltpu.VMEM((CHUNK,), jnp.int32),
            pltpu.VMEM((CHUNK, N_HID), jnp.float32),
            pltpu.VMEM((N_HID, CHUNK), jnp.float32),
            pltpu.SemaphoreType.DMA,
        ],
        compiler_params=pltpu.CompilerParams(use_tc_tiling_on_sc=False,
                                             needs_layout_passes=False),
    )
    def gather_k(table_hbm, idx_hbm, tt_hbm, out_hbm, idx_v, rows_v, tt_v, sem):
        wid = lax.axis_index("s") * NC + lax.axis_index("c")
        lane = lax.iota(jnp.int32, 16)
        for j in range(N_CHUNKS):
            base = wid * E_PER_W + j * CHUNK
            pltpu.sync_copy(idx_hbm.at[pl.ds(base, CHUNK)], idx_v)
            cp = pltpu.async_copy(table_hbm.at[idx_v], rows_v, sem)
            pltpu.sync_copy(tt_hbm.at[:, pl.ds(base, CHUNK)], tt_v)
            cp.wait()

            def add_t(o, _):
                for i in range(16):
                    e = o * 16 + i
                    tcol = plsc.load_gather(
                        tt_v, [lane, jnp.full((16,), e, jnp.int32)])
                    rows_v[e] = rows_v[e] + tcol
                return 0

            lax.fori_loop(0, CHUNK // 16, add_t, 0)
            pltpu.sync_copy(rows_v, out_hbm.at[pl.ds(base, CHUNK)])

    return gather_k


_gather_cache = []


def _sc_gather(table, idx, tt):
    if not _gather_cache:
        _gather_cache.append(_sc_gather_build())
    return _gather_cache[0](table, idx, tt)


def _sc_scatter_build():
    mesh = plsc.VectorSubcoreMesh(core_axis_name="c", subcore_axis_name="s")

    @functools.partial(
        pl.kernel,
        mesh=mesh,
        out_type=(
            jax.ShapeDtypeStruct((NC, NPAD, N_HID), jnp.float32),
            jax.ShapeDtypeStruct((NC, NPAD), jnp.float32),
        ),
        scratch_types=[
            pltpu.VMEM_SHARED((NPAD, N_HID), jnp.float32),
            pltpu.VMEM_SHARED((NPAD,), jnp.float32),
            pltpu.VMEM((CHUNK,), jnp.int32),
            pltpu.VMEM((CHUNK, N_HID), jnp.float32),
            pltpu.VMEM((CHUNK,), jnp.float32),
            pltpu.VMEM((STRIPE, N_HID), jnp.float32),
            pltpu.VMEM((STRIPE,), jnp.float32),
        ],
        compiler_params=pltpu.CompilerParams(use_tc_tiling_on_sc=False),
    )
    def scatter_k(row_hbm, h_hbm, acc_out, cnt_out,
                  acc_s, cnt_s, idx_v, h_v, ones_v, zrow_v, zcnt_v):
        c = lax.axis_index("c")
        s = lax.axis_index("s")

        def fill_rows(i, _):
            zrow_v[i] = jnp.zeros((N_HID,), jnp.float32)
            return 0

        lax.fori_loop(0, STRIPE, fill_rows, 0)

        def fill_1d(i, _):
            zcnt_v[pl.ds(i * 16, 16)] = jnp.zeros((16,), jnp.float32)
            ones_v[pl.ds(i * 16, 16)] = jnp.ones((16,), jnp.float32)
            return 0

        lax.fori_loop(0, STRIPE // 16, fill_1d, 0)

        def fill_ones_tail(i, _):
            ones_v[pl.ds(i * 16, 16)] = jnp.ones((16,), jnp.float32)
            return 0

        lax.fori_loop(STRIPE // 16, CHUNK // 16, fill_ones_tail, 0)

        pltpu.sync_copy(zrow_v, acc_s.at[pl.ds(s * STRIPE, STRIPE)])
        pltpu.sync_copy(zcnt_v, cnt_s.at[pl.ds(s * STRIPE, STRIPE)])
        plsc.subcore_barrier()

        for j in range(N_CHUNKS):
            base = c * (N_EDGES // NC) + s * E_PER_W + j * CHUNK
            pltpu.sync_copy(row_hbm.at[pl.ds(base, CHUNK)], idx_v)
            pltpu.sync_copy(h_hbm.at[pl.ds(base, CHUNK)], h_v)
            pltpu.sync_copy(h_v, acc_s.at[idx_v], add=True)
            pltpu.sync_copy(ones_v, cnt_s.at[idx_v], add=True)

        plsc.subcore_barrier()
        pltpu.sync_copy(acc_s.at[pl.ds(s * STRIPE, STRIPE)],
                        acc_out.at[c, pl.ds(s * STRIPE, STRIPE)])
        pltpu.sync_copy(cnt_s.at[pl.ds(s * STRIPE, STRIPE)],
                        cnt_out.at[c, pl.ds(s * STRIPE, STRIPE)])

    return scatter_k


_scatter_cache = []


def _sc_scatter(row, h):
    if not _scatter_cache:
        _scatter_cache.append(_sc_scatter_build())
    return _scatter_cache[0](row, h)


def _edge_body(g_ref, w2c_ref, s16_ref, b2c_ref, g1_ref, be1_ref, o_ref):
    r = jnp.maximum(g_ref[...], 0.0)
    cc = jnp.dot(r, w2c_ref[...]) + b2c_ref[...]
    v = jnp.dot(cc * cc, s16_ref[...])
    o_ref[...] = cc * jax.lax.rsqrt(v + EPS) * g1_ref[...] + be1_ref[...]


def _tc_edge(g_p, w2c_blk, s16, b2c_t, g1_t, be1_t):
    grid = 8
    eb = EP // grid
    row_spec = pl.BlockSpec((eb, 128), lambda i: (i, 0))
    full = pl.BlockSpec((128, 128), lambda i: (0, 0))
    vec = pl.BlockSpec((1, 128), lambda i: (0, 0))
    return pl.pallas_call(
        _edge_body,
        grid=(grid,),
        in_specs=[row_spec, full, full, vec, vec, vec],
        out_specs=row_spec,
        out_shape=jax.ShapeDtypeStruct((EP, 128), jnp.float32),
    )(g_p, w2c_blk, s16, b2c_t, g1_t, be1_t)


def _node_body(xr_ref, acc_ref, cnt_ref, u_ref, w3x_ref, w3m_ref, w3u_ref,
               b8_ref, w4c_ref, s16_ref, b3_ref, b4c_ref, g2_ref, be2_ref,
               o_ref):
    acc = acc_ref[0][:NP] + acc_ref[1][:NP]
    cntn = cnt_ref[0][:NP] + cnt_ref[1][:NP]
    cnt_p = jnp.dot(cntn, b8_ref[...])
    mean_p = acc / jnp.maximum(cnt_p, 1.0)
    z = (jnp.dot(xr_ref[...], w3x_ref[...])
         + jnp.dot(mean_p, w3m_ref[...])
         + jnp.dot(u_ref[...], w3u_ref[...])
         + b3_ref[...])
    r = jnp.maximum(z, 0.0)
    cc = jnp.dot(r, w4c_ref[...]) + b4c_ref[...]
    v = jnp.dot(cc * cc, s16_ref[...])
    o_ref[...] = cc * jax.lax.rsqrt(v + EPS) * g2_ref[...] + be2_ref[...]


def _tc_node(xr, acc_p, cnt_r, u_p, w3x_big, w3m_blk, w3u_blk, b8, w4c_blk,
             s16, b3_t, b4c_t, g2_t, be2_t):
    return pl.pallas_call(
        _node_body,
        out_shape=jax.ShapeDtypeStruct((NP, 128), jnp.float32),
    )(xr, acc_p, cnt_r, u_p, w3x_big, w3m_blk, w3u_blk, b8, w4c_blk,
      s16, b3_t, b4c_t, g2_t, be2_t)


def kernel(x, edge_idx, edge_attr, u, W1, b1, W2, b2, g1, be1,
           W3, b3, W4, b4, g2, be2):
    row = edge_idx[0].astype(jnp.int32)
    col = edge_idx[1].astype(jnp.int32)

    i8 = jnp.eye(PACK, dtype=jnp.float32)
    w1x_big = jnp.kron(i8, W1[:N_FEAT])
    w1t = W1[N_FEAT:].T
    w2c = W2 - jnp.mean(W2, axis=1, keepdims=True)
    w2c_blk = jnp.kron(i8, w2c)
    w4c = W4 - jnp.mean(W4, axis=1, keepdims=True)
    w4c_blk = jnp.kron(i8, w4c)
    w3x_big = jnp.kron(i8, W3[:N_FEAT])
    w3m_blk = jnp.kron(i8, W3[N_FEAT:N_FEAT + N_HID])
    w3u_blk = jnp.kron(i8, W3[N_FEAT + N_HID:])
    s16 = jnp.kron(i8, jnp.full((N_HID, N_HID), 1.0 / N_HID, jnp.float32))
    b8 = jnp.kron(i8, jnp.ones((1, N_HID), jnp.float32))

    def tile8(v):
        return jnp.tile(v, PACK).reshape(1, 128)

    b1c = b1.reshape(N_HID, 1)
    b2c_t = tile8(b2 - jnp.mean(b2))
    b4c_t = tile8(b4 - jnp.mean(b4))
    b3_t = tile8(b3)
    g1_t, be1_t = tile8(g1), tile8(be1)
    g2_t, be2_t = tile8(g2), tile8(be2)

    xr = x.reshape(NP, PACK * N_FEAT)

    xw1 = _tc_xw(xr, w1x_big).reshape(N_NODES, N_HID)

    tt = _tc_tt(w1t, edge_attr.T, b1c)

    g = _sc_gather(xw1, col, tt)

    h_p = _tc_edge(g.reshape(EP, 128), w2c_blk, s16, b2c_t, g1_t, be1_t)
    h = h_p.reshape(N_EDGES, N_HID)

    acc2, cnt2 = _sc_scatter(row, h)

    acc_p = acc2.reshape(NC, NPP, 128)
    cnt_r = cnt2.reshape(NC, NPP, PACK)
    u_p = u.reshape(NP, 128)
    out_p = _tc_node(xr, acc_p, cnt_r, u_p, w3x_big, w3m_blk, w3u_blk, b8,
                     w4c_blk, s16, b3_t, b4c_t, g2_t, be2_t)
    return out_p.reshape(N_NODES, N_HID)


# speedup vs baseline: 14.3258x; 1.6212x over previous
import functools

import jax
import jax.numpy as jnp
from jax import lax
from jax.experimental import pallas as pl
from jax.experimental.pallas import tpu as pltpu
from jax.experimental.pallas import tpu_sc as plsc

N_NODES = 10000
N_EDGES = 320000
N_FEAT = 128
N_HID = 16
PACK = 128 // N_HID
EP = N_EDGES // PACK
NP = N_NODES // PACK
EPS = 1e-5

NC, NS = 2, 16
NW = NC * NS
E_PER_W = N_EDGES // NW
CHUNK = 2000
N_CHUNKS = E_PER_W // CHUNK
NPAD = 10240
NPP = NPAD // PACK
STRIPE = NPAD // NS


def _xw_body(xr_ref, w_ref, o_ref):
    o_ref[...] = jnp.dot(xr_ref[...], w_ref[...])


def _tc_xw(xr, wbig):
    return pl.pallas_call(
        _xw_body,
        out_shape=jax.ShapeDtypeStruct((NP, 128), jnp.float32),
    )(xr, wbig)


def _tt_body(w_ref, ea_ref, b_ref, o_ref):
    o_ref[...] = jnp.dot(w_ref[...], ea_ref[...]) + b_ref[...]


def _tc_tt(w1t, eaT, b1c):
    grid = 10
    eb = N_EDGES // grid
    col_spec = pl.BlockSpec((N_HID, eb), lambda i: (0, i))
    return pl.pallas_call(
        _tt_body,
        grid=(grid,),
        in_specs=[pl.BlockSpec((N_HID, N_HID), lambda i: (0, 0)),
                  col_spec,
                  pl.BlockSpec((N_HID, 1), lambda i: (0, 0))],
        out_specs=col_spec,
        out_shape=jax.ShapeDtypeStruct((N_HID, N_EDGES), jnp.float32),
    )(w1t, eaT, b1c)


def _sc_gather_build():
    mesh = plsc.VectorSubcoreMesh(core_axis_name="c", subcore_axis_name="s")

    @functools.partial(
        pl.kernel,
        mesh=mesh,
        out_type=jax.ShapeDtypeStruct((N_EDGES, N_HID), jnp.float32),
        scratch_types=[
            pltpu.VMEM((2, CHUNK), jnp.int32),
            pltpu.VMEM((2, CHUNK, N_HID), jnp.float32),
            pltpu.VMEM((N_HID, CHUNK + 8), jnp.float32),
            pltpu.SemaphoreType.DMA,
            pltpu.SemaphoreType.DMA,
            pltpu.SemaphoreType.DMA,
            pltpu.SemaphoreType.DMA,
        ],
        compiler_params=pltpu.CompilerParams(use_tc_tiling_on_sc=False,
                                             needs_layout_passes=False),
    )
    def gather_k(table_hbm, eidx_hbm, tt_hbm, out_hbm,
                 idx_v, rows_v, tt_v, sg0, sg1, so0, so1):
        wid = lax.axis_index("s") * NC + lax.axis_index("c")
        lane = lax.iota(jnp.int32, 16)
        sgs = (sg0, sg1)
        sos = (so0, so1)

        def ebase(j):
            return wid * E_PER_W + j * CHUNK

        pltpu.sync_copy(eidx_hbm.at[1, pl.ds(ebase(0), CHUNK)], idx_v.at[0])
        gcp = [None, None]
        ocp = [None, None]
        gcp[0] = pltpu.async_copy(table_hbm.at[idx_v.at[0]], rows_v.at[0],
                                  sgs[0])
        for j in range(N_CHUNKS):
            b = j % 2
            nb = 1 - b
            if j + 1 < N_CHUNKS:
                pltpu.sync_copy(eidx_hbm.at[1, pl.ds(ebase(j + 1), CHUNK)],
                                idx_v.at[nb])
                if ocp[nb] is not None:
                    ocp[nb].wait()
                    ocp[nb] = None
                gcp[nb] = pltpu.async_copy(table_hbm.at[idx_v.at[nb]],
                                           rows_v.at[nb], sgs[nb])
            pltpu.sync_copy(tt_hbm.at[:, pl.ds(ebase(j), CHUNK)],
                            tt_v.at[:, pl.ds(0, CHUNK)])
            gcp[b].wait()

            def add_t(o, _):
                base_vec = jnp.full((16,), o * 16, jnp.int32)
                tc = [None] * 16
                for i in range(16):
                    tc[i] = plsc.load_gather(tt_v, [lane, base_vec + i])
                for i in range(16):
                    e = o * 16 + i
                    rows_v[b, e] = rows_v[b, e] + tc[i]
                return 0

            lax.fori_loop(0, CHUNK // 16, add_t, 0)
            ocp[b] = pltpu.async_copy(rows_v.at[b],
                                      out_hbm.at[pl.ds(ebase(j), CHUNK)],
                                      sos[b])
        for b in range(2):
            if ocp[b] is not None:
                ocp[b].wait()

    return gather_k


_gather_cache = []


def _sc_gather(table, idx, tt):
    if not _gather_cache:
        _gather_cache.append(_sc_gather_build())
    return _gather_cache[0](table, idx, tt)


def _sc_scatter_build():
    mesh = plsc.VectorSubcoreMesh(core_axis_name="c", subcore_axis_name="s")

    @functools.partial(
        pl.kernel,
        mesh=mesh,
        out_type=(
            jax.ShapeDtypeStruct((NC, NPAD, N_HID), jnp.float32),
            jax.ShapeDtypeStruct((NC, NPAD), jnp.float32),
        ),
        scratch_types=[
            pltpu.VMEM_SHARED((NPAD, N_HID), jnp.float32),
            pltpu.VMEM_SHARED((NPAD,), jnp.float32),
            pltpu.VMEM((2, CHUNK), jnp.int32),
            pltpu.VMEM((2, CHUNK, N_HID), jnp.float32),
            pltpu.VMEM((CHUNK,), jnp.float32),
            pltpu.VMEM((STRIPE, N_HID), jnp.float32),
            pltpu.VMEM((STRIPE,), jnp.float32),
            pltpu.SemaphoreType.DMA,
            pltpu.SemaphoreType.DMA,
            pltpu.SemaphoreType.DMA,
            pltpu.SemaphoreType.DMA,
        ],
        compiler_params=pltpu.CompilerParams(use_tc_tiling_on_sc=False),
    )
    def scatter_k(eidx_hbm, h_hbm, acc_out, cnt_out,
                  acc_s, cnt_s, idx_v, h_v, ones_v, zrow_v, zcnt_v,
                  si0, si1, sh0, sh1):
        c = lax.axis_index("c")
        s = lax.axis_index("s")
        sis = (si0, si1)
        shs = (sh0, sh1)

        def cbase(j):
            return c * (N_EDGES // NC) + s * E_PER_W + j * CHUNK

        def start_loads(j, bb):
            icp = pltpu.async_copy(eidx_hbm.at[0, pl.ds(cbase(j), CHUNK)],
                                   idx_v.at[bb], sis[bb])
            hcp = pltpu.async_copy(h_hbm.at[pl.ds(cbase(j), CHUNK)],
                                   h_v.at[bb], shs[bb])
            return icp, hcp

        cps = [start_loads(0, 0), None]

        def fill_rows(i, _):
            zrow_v[i] = jnp.zeros((N_HID,), jnp.float32)
            return 0

        lax.fori_loop(0, STRIPE, fill_rows, 0)

        def fill_1d(i, _):
            zcnt_v[pl.ds(i * 16, 16)] = jnp.zeros((16,), jnp.float32)
            ones_v[pl.ds(i * 16, 16)] = jnp.ones((16,), jnp.float32)
            return 0

        lax.fori_loop(0, STRIPE // 16, fill_1d, 0)

        def fill_ones_tail(i, _):
            ones_v[pl.ds(i * 16, 16)] = jnp.ones((16,), jnp.float32)
            return 0

        lax.fori_loop(STRIPE // 16, CHUNK // 16, fill_ones_tail, 0)

        pltpu.sync_copy(zrow_v, acc_s.at[pl.ds(s * STRIPE, STRIPE)])
        pltpu.sync_copy(zcnt_v, cnt_s.at[pl.ds(s * STRIPE, STRIPE)])
        plsc.subcore_barrier()

        for j in range(N_CHUNKS):
            b = j % 2
            if j + 1 < N_CHUNKS:
                cps[1 - b] = start_loads(j + 1, 1 - b)
            icp, hcp = cps[b]
            icp.wait()
            hcp.wait()
            pltpu.sync_copy(h_v.at[b], acc_s.at[idx_v.at[b]], add=True)
            pltpu.sync_copy(ones_v, cnt_s.at[idx_v.at[b]], add=True)

        plsc.subcore_barrier()
        pltpu.sync_copy(acc_s.at[pl.ds(s * STRIPE, STRIPE)],
                        acc_out.at[c, pl.ds(s * STRIPE, STRIPE)])
        pltpu.sync_copy(cnt_s.at[pl.ds(s * STRIPE, STRIPE)],
                        cnt_out.at[c, pl.ds(s * STRIPE, STRIPE)])

    return scatter_k


_scatter_cache = []


def _sc_scatter(row, h):
    if not _scatter_cache:
        _scatter_cache.append(_sc_scatter_build())
    return _scatter_cache[0](row, h)


def _edge_body(g_ref, w2c_ref, s16_ref, b2c_ref, g1_ref, be1_ref, o_ref):
    r = jnp.maximum(g_ref[...], 0.0)
    cc = jnp.dot(r, w2c_ref[...]) + b2c_ref[...]
    v = jnp.dot(cc * cc, s16_ref[...])
    o_ref[...] = cc * jax.lax.rsqrt(v + EPS) * g1_ref[...] + be1_ref[...]


def _tc_edge(g_p, w2c_blk, s16, b2c_t, g1_t, be1_t):
    grid = 8
    eb = EP // grid
    row_spec = pl.BlockSpec((eb, 128), lambda i: (i, 0))
    full = pl.BlockSpec((128, 128), lambda i: (0, 0))
    vec = pl.BlockSpec((1, 128), lambda i: (0, 0))
    return pl.pallas_call(
        _edge_body,
        grid=(grid,),
        in_specs=[row_spec, full, full, vec, vec, vec],
        out_specs=row_spec,
        out_shape=jax.ShapeDtypeStruct((EP, 128), jnp.float32),
    )(g_p, w2c_blk, s16, b2c_t, g1_t, be1_t)


def _node_body(xr_ref, acc_ref, cnt_ref, u_ref, w3x_ref, w3m_ref, w3u_ref,
               b8_ref, w4c_ref, s16_ref, b3_ref, b4c_ref, g2_ref, be2_ref,
               o_ref):
    acc = acc_ref[0][:NP] + acc_ref[1][:NP]
    cntn = cnt_ref[0][:NP] + cnt_ref[1][:NP]
    cnt_p = jnp.dot(cntn, b8_ref[...])
    mean_p = acc / jnp.maximum(cnt_p, 1.0)
    z = (jnp.dot(xr_ref[...], w3x_ref[...])
         + jnp.dot(mean_p, w3m_ref[...])
         + jnp.dot(u_ref[...], w3u_ref[...])
         + b3_ref[...])
    r = jnp.maximum(z, 0.0)
    cc = jnp.dot(r, w4c_ref[...]) + b4c_ref[...]
    v = jnp.dot(cc * cc, s16_ref[...])
    o_ref[...] = cc * jax.lax.rsqrt(v + EPS) * g2_ref[...] + be2_ref[...]


def _tc_node(xr, acc_p, cnt_r, u_p, w3x_big, w3m_blk, w3u_blk, b8, w4c_blk,
             s16, b3_t, b4c_t, g2_t, be2_t):
    return pl.pallas_call(
        _node_body,
        out_shape=jax.ShapeDtypeStruct((NP, 128), jnp.float32),
    )(xr, acc_p, cnt_r, u_p, w3x_big, w3m_blk, w3u_blk, b8, w4c_blk,
      s16, b3_t, b4c_t, g2_t, be2_t)


def kernel(x, edge_idx, edge_attr, u, W1, b1, W2, b2, g1, be1,
           W3, b3, W4, b4, g2, be2):
    eidx = edge_idx.astype(jnp.int32)

    i8 = jnp.eye(PACK, dtype=jnp.float32)
    w1x_big = jnp.kron(i8, W1[:N_FEAT])
    w1t = W1[N_FEAT:].T
    w2c = W2 - jnp.mean(W2, axis=1, keepdims=True)
    w2c_blk = jnp.kron(i8, w2c)
    w4c = W4 - jnp.mean(W4, axis=1, keepdims=True)
    w4c_blk = jnp.kron(i8, w4c)
    w3x_big = jnp.kron(i8, W3[:N_FEAT])
    w3m_blk = jnp.kron(i8, W3[N_FEAT:N_FEAT + N_HID])
    w3u_blk = jnp.kron(i8, W3[N_FEAT + N_HID:])
    s16 = jnp.kron(i8, jnp.full((N_HID, N_HID), 1.0 / N_HID, jnp.float32))
    b8 = jnp.kron(i8, jnp.ones((1, N_HID), jnp.float32))

    def tile8(v):
        return jnp.tile(v, PACK).reshape(1, 128)

    b1c = b1.reshape(N_HID, 1)
    b2c_t = tile8(b2 - jnp.mean(b2))
    b4c_t = tile8(b4 - jnp.mean(b4))
    b3_t = tile8(b3)
    g1_t, be1_t = tile8(g1), tile8(be1)
    g2_t, be2_t = tile8(g2), tile8(be2)

    xr = x.reshape(NP, PACK * N_FEAT)

    xw1 = _tc_xw(xr, w1x_big).reshape(N_NODES, N_HID)

    tt = _tc_tt(w1t, edge_attr.T, b1c)

    g = _sc_gather(xw1, eidx, tt)

    h_p = _tc_edge(g.reshape(EP, 128), w2c_blk, s16, b2c_t, g1_t, be1_t)
    h = h_p.reshape(N_EDGES, N_HID)

    acc2, cnt2 = _sc_scatter(eidx, h)

    u_late, _ = lax.optimization_barrier((u, g))

    acc_p = acc2.reshape(NC, NPP, 128)
    cnt_r = cnt2.reshape(NC, NPP, PACK)
    u_p = u_late.reshape(NP, 128)
    out_p = _tc_node(xr, acc_p, cnt_r, u_p, w3x_big, w3m_blk, w3u_blk, b8,
                     w4c_blk, s16, b3_t, b4c_t, g2_t, be2_t)
    return out_p.reshape(N_NODES, N_HID)


# speedup vs baseline: 14.8390x; 1.0358x over previous
import functools

import jax
import jax.numpy as jnp
from jax import lax
from jax.experimental import pallas as pl
from jax.experimental.pallas import tpu as pltpu
from jax.experimental.pallas import tpu_sc as plsc

N_NODES = 10000
N_EDGES = 320000
N_FEAT = 128
N_HID = 16
PACK = 128 // N_HID
EP = N_EDGES // PACK
NP = N_NODES // PACK
EPS = 1e-5

NC, NS = 2, 16
NW = NC * NS
E_PER_W = N_EDGES // NW
CHUNK = 2000
N_CHUNKS = E_PER_W // CHUNK
NPAD = 10240
NPP = NPAD // PACK
STRIPE = NPAD // NS


def _xw_body(xr_ref, w_ref, o_ref):
    o_ref[...] = jnp.dot(xr_ref[...], w_ref[...])


def _tc_xw(xr, wbig):
    return pl.pallas_call(
        _xw_body,
        out_shape=jax.ShapeDtypeStruct((NP, 128), jnp.float32),
    )(xr, wbig)


def _tt_body(w_ref, ea_ref, b_ref, o_ref):
    o_ref[...] = jnp.dot(w_ref[...], ea_ref[...]) + b_ref[...]


def _tc_tt(w1t, eaT, b1c):
    grid = 5
    eb = N_EDGES // grid
    col_spec = pl.BlockSpec((N_HID, eb), lambda i: (0, i))
    return pl.pallas_call(
        _tt_body,
        grid=(grid,),
        in_specs=[pl.BlockSpec((N_HID, N_HID), lambda i: (0, 0)),
                  col_spec,
                  pl.BlockSpec((N_HID, 1), lambda i: (0, 0))],
        out_specs=col_spec,
        out_shape=jax.ShapeDtypeStruct((N_HID, N_EDGES), jnp.float32),
    )(w1t, eaT, b1c)


def _sc_gather_build():
    mesh = plsc.VectorSubcoreMesh(core_axis_name="c", subcore_axis_name="s")

    @functools.partial(
        pl.kernel,
        mesh=mesh,
        out_type=jax.ShapeDtypeStruct((N_EDGES, N_HID), jnp.float32),
        scratch_types=[
            pltpu.VMEM((2, CHUNK), jnp.int32),
            pltpu.VMEM((2, CHUNK, N_HID), jnp.float32),
            pltpu.VMEM((N_HID, CHUNK + 8), jnp.float32),
            pltpu.SemaphoreType.DMA,
            pltpu.SemaphoreType.DMA,
            pltpu.SemaphoreType.DMA,
            pltpu.SemaphoreType.DMA,
        ],
        compiler_params=pltpu.CompilerParams(use_tc_tiling_on_sc=False,
                                             needs_layout_passes=False),
    )
    def gather_k(table_hbm, eidx_hbm, tt_hbm, out_hbm,
                 idx_v, rows_v, tt_v, sg0, sg1, so0, so1):
        wid = lax.axis_index("s") * NC + lax.axis_index("c")
        lane = lax.iota(jnp.int32, 16)
        sgs = (sg0, sg1)
        sos = (so0, so1)

        def ebase(j):
            return wid * E_PER_W + j * CHUNK

        pltpu.sync_copy(eidx_hbm.at[1, pl.ds(ebase(0), CHUNK)], idx_v.at[0])
        gcp = [None, None]
        ocp = [None, None]
        gcp[0] = pltpu.async_copy(table_hbm.at[idx_v.at[0]], rows_v.at[0],
                                  sgs[0])
        for j in range(N_CHUNKS):
            b = j % 2
            nb = 1 - b
            if j + 1 < N_CHUNKS:
                pltpu.sync_copy(eidx_hbm.at[1, pl.ds(ebase(j + 1), CHUNK)],
                                idx_v.at[nb])
                if ocp[nb] is not None:
                    ocp[nb].wait()
                    ocp[nb] = None
                gcp[nb] = pltpu.async_copy(table_hbm.at[idx_v.at[nb]],
                                           rows_v.at[nb], sgs[nb])
            pltpu.sync_copy(tt_hbm.at[:, pl.ds(ebase(j), CHUNK)],
                            tt_v.at[:, pl.ds(0, CHUNK)])
            gcp[b].wait()

            def add_t(o, _):
                base_vec = jnp.full((16,), o * 16, jnp.int32)
                tc = [None] * 16
                for i in range(16):
                    tc[i] = plsc.load_gather(tt_v, [lane, base_vec + i])
                for i in range(16):
                    e = o * 16 + i
                    rows_v[b, e] = rows_v[b, e] + tc[i]
                return 0

            lax.fori_loop(0, CHUNK // 16, add_t, 0)
            ocp[b] = pltpu.async_copy(rows_v.at[b],
                                      out_hbm.at[pl.ds(ebase(j), CHUNK)],
                                      sos[b])
        for b in range(2):
            if ocp[b] is not None:
                ocp[b].wait()

    return gather_k


_gather_cache = []


def _sc_gather(table, idx, tt):
    if not _gather_cache:
        _gather_cache.append(_sc_gather_build())
    return _gather_cache[0](table, idx, tt)


def _sc_scatter_build():
    mesh = plsc.VectorSubcoreMesh(core_axis_name="c", subcore_axis_name="s")

    @functools.partial(
        pl.kernel,
        mesh=mesh,
        out_type=(
            jax.ShapeDtypeStruct((NC, NPAD, N_HID), jnp.float32),
            jax.ShapeDtypeStruct((NC, NPAD), jnp.float32),
        ),
        scratch_types=[
            pltpu.VMEM_SHARED((NPAD, N_HID), jnp.float32),
            pltpu.VMEM_SHARED((NPAD,), jnp.float32),
            pltpu.VMEM((2, CHUNK), jnp.int32),
            pltpu.VMEM((2, CHUNK, N_HID), jnp.float32),
            pltpu.VMEM((CHUNK,), jnp.float32),
            pltpu.VMEM((STRIPE, N_HID), jnp.float32),
            pltpu.VMEM((STRIPE,), jnp.float32),
            pltpu.SemaphoreType.DMA,
            pltpu.SemaphoreType.DMA,
            pltpu.SemaphoreType.DMA,
            pltpu.SemaphoreType.DMA,
        ],
        compiler_params=pltpu.CompilerParams(use_tc_tiling_on_sc=False),
    )
    def scatter_k(eidx_hbm, h_hbm, acc_out, cnt_out,
                  acc_s, cnt_s, idx_v, h_v, ones_v, zrow_v, zcnt_v,
                  si0, si1, sh0, sh1):
        c = lax.axis_index("c")
        s = lax.axis_index("s")
        sis = (si0, si1)
        shs = (sh0, sh1)

        def cbase(j):
            return c * (N_EDGES // NC) + s * E_PER_W + j * CHUNK

        def start_loads(j, bb):
            icp = pltpu.async_copy(eidx_hbm.at[0, pl.ds(cbase(j), CHUNK)],
                                   idx_v.at[bb], sis[bb])
            hcp = pltpu.async_copy(h_hbm.at[pl.ds(cbase(j), CHUNK)],
                                   h_v.at[bb], shs[bb])
            return icp, hcp

        cps = [start_loads(0, 0), None]

        def fill_rows(i, _):
            zrow_v[i] = jnp.zeros((N_HID,), jnp.float32)
            return 0

        lax.fori_loop(0, STRIPE, fill_rows, 0)

        def fill_1d(i, _):
            zcnt_v[pl.ds(i * 16, 16)] = jnp.zeros((16,), jnp.float32)
            ones_v[pl.ds(i * 16, 16)] = jnp.ones((16,), jnp.float32)
            return 0

        lax.fori_loop(0, STRIPE // 16, fill_1d, 0)

        def fill_ones_tail(i, _):
            ones_v[pl.ds(i * 16, 16)] = jnp.ones((16,), jnp.float32)
            return 0

        lax.fori_loop(STRIPE // 16, CHUNK // 16, fill_ones_tail, 0)

        pltpu.sync_copy(zrow_v, acc_s.at[pl.ds(s * STRIPE, STRIPE)])
        pltpu.sync_copy(zcnt_v, cnt_s.at[pl.ds(s * STRIPE, STRIPE)])
        plsc.subcore_barrier()

        for j in range(N_CHUNKS):
            b = j % 2
            if j + 1 < N_CHUNKS:
                cps[1 - b] = start_loads(j + 1, 1 - b)
            icp, hcp = cps[b]
            icp.wait()
            hcp.wait()
            pltpu.sync_copy(h_v.at[b], acc_s.at[idx_v.at[b]], add=True)
            pltpu.sync_copy(ones_v, cnt_s.at[idx_v.at[b]], add=True)

        plsc.subcore_barrier()
        pltpu.sync_copy(acc_s.at[pl.ds(s * STRIPE, STRIPE)],
                        acc_out.at[c, pl.ds(s * STRIPE, STRIPE)])
        pltpu.sync_copy(cnt_s.at[pl.ds(s * STRIPE, STRIPE)],
                        cnt_out.at[c, pl.ds(s * STRIPE, STRIPE)])

    return scatter_k


_scatter_cache = []


def _sc_scatter(row, h):
    if not _scatter_cache:
        _scatter_cache.append(_sc_scatter_build())
    return _scatter_cache[0](row, h)


def _edge_body(g_ref, w2c_ref, s16_ref, b2c_ref, g1_ref, be1_ref, o_ref):
    r = jnp.maximum(g_ref[...], 0.0)
    cc = jnp.dot(r, w2c_ref[...]) + b2c_ref[...]
    v = jnp.dot(cc * cc, s16_ref[...])
    o_ref[...] = cc * jax.lax.rsqrt(v + EPS) * g1_ref[...] + be1_ref[...]


def _tc_edge(g_p, w2c_blk, s16, b2c_t, g1_t, be1_t):
    grid = 4
    eb = EP // grid
    row_spec = pl.BlockSpec((eb, 128), lambda i: (i, 0))
    full = pl.BlockSpec((128, 128), lambda i: (0, 0))
    vec = pl.BlockSpec((1, 128), lambda i: (0, 0))
    return pl.pallas_call(
        _edge_body,
        grid=(grid,),
        in_specs=[row_spec, full, full, vec, vec, vec],
        out_specs=row_spec,
        out_shape=jax.ShapeDtypeStruct((EP, 128), jnp.float32),
    )(g_p, w2c_blk, s16, b2c_t, g1_t, be1_t)


def _node_body(xr_ref, acc_ref, cnt_ref, u_ref, w3x_ref, w3m_ref, w3u_ref,
               b8_ref, w4c_ref, s16_ref, b3_ref, b4c_ref, g2_ref, be2_ref,
               o_ref):
    acc = acc_ref[0][:NP] + acc_ref[1][:NP]
    cntn = cnt_ref[0][:NP] + cnt_ref[1][:NP]
    cnt_p = jnp.dot(cntn, b8_ref[...])
    mean_p = acc / jnp.maximum(cnt_p, 1.0)
    z = (jnp.dot(xr_ref[...], w3x_ref[...])
         + jnp.dot(mean_p, w3m_ref[...])
         + jnp.dot(u_ref[...], w3u_ref[...])
         + b3_ref[...])
    r = jnp.maximum(z, 0.0)
    cc = jnp.dot(r, w4c_ref[...]) + b4c_ref[...]
    v = jnp.dot(cc * cc, s16_ref[...])
    o_ref[...] = cc * jax.lax.rsqrt(v + EPS) * g2_ref[...] + be2_ref[...]


def _tc_node(xr, acc_p, cnt_r, u_p, w3x_big, w3m_blk, w3u_blk, b8, w4c_blk,
             s16, b3_t, b4c_t, g2_t, be2_t):
    return pl.pallas_call(
        _node_body,
        out_shape=jax.ShapeDtypeStruct((NP, 128), jnp.float32),
    )(xr, acc_p, cnt_r, u_p, w3x_big, w3m_blk, w3u_blk, b8, w4c_blk,
      s16, b3_t, b4c_t, g2_t, be2_t)


def kernel(x, edge_idx, edge_attr, u, W1, b1, W2, b2, g1, be1,
           W3, b3, W4, b4, g2, be2):
    eidx = edge_idx.astype(jnp.int32)

    i8 = jnp.eye(PACK, dtype=jnp.float32)
    w1x_big = jnp.kron(i8, W1[:N_FEAT])
    w1t = W1[N_FEAT:].T
    w2c = W2 - jnp.mean(W2, axis=1, keepdims=True)
    w2c_blk = jnp.kron(i8, w2c)
    w4c = W4 - jnp.mean(W4, axis=1, keepdims=True)
    w4c_blk = jnp.kron(i8, w4c)
    w3x_big = jnp.kron(i8, W3[:N_FEAT])
    w3m_blk = jnp.kron(i8, W3[N_FEAT:N_FEAT + N_HID])
    w3u_blk = jnp.kron(i8, W3[N_FEAT + N_HID:])
    s16 = jnp.kron(i8, jnp.full((N_HID, N_HID), 1.0 / N_HID, jnp.float32))
    b8 = jnp.kron(i8, jnp.ones((1, N_HID), jnp.float32))

    def tile8(v):
        return jnp.tile(v, PACK).reshape(1, 128)

    b1c = b1.reshape(N_HID, 1)
    b2c_t = tile8(b2 - jnp.mean(b2))
    b4c_t = tile8(b4 - jnp.mean(b4))
    b3_t = tile8(b3)
    g1_t, be1_t = tile8(g1), tile8(be1)
    g2_t, be2_t = tile8(g2), tile8(be2)

    xr = x.reshape(NP, PACK * N_FEAT)

    xw1 = _tc_xw(xr, w1x_big).reshape(N_NODES, N_HID)

    tt = _tc_tt(w1t, edge_attr.T, b1c)

    g = _sc_gather(xw1, eidx, tt)

    h_p = _tc_edge(g.reshape(EP, 128), w2c_blk, s16, b2c_t, g1_t, be1_t)
    h = h_p.reshape(N_EDGES, N_HID)

    acc2, cnt2 = _sc_scatter(eidx, h)

    u_late, _ = lax.optimization_barrier((u, g))

    acc_p = acc2.reshape(NC, NPP, 128)
    cnt_r = cnt2.reshape(NC, NPP, PACK)
    u_p = u_late.reshape(NP, 128)
    out_p = _tc_node(xr, acc_p, cnt_r, u_p, w3x_big, w3m_blk, w3u_blk, b8,
                     w4c_blk, s16, b3_t, b4c_t, g2_t, be2_t)
    return out_p.reshape(N_NODES, N_HID)


# speedup vs baseline: 15.1233x; 1.0192x over previous
import functools

import jax
import jax.numpy as jnp
from jax import lax
from jax.experimental import pallas as pl
from jax.experimental.pallas import tpu as pltpu
from jax.experimental.pallas import tpu_sc as plsc

N_NODES = 10000
N_EDGES = 320000
N_FEAT = 128
N_HID = 16
PACK = 128 // N_HID
EP = N_EDGES // PACK
NP = N_NODES // PACK
EPS = 1e-5

NC, NS = 2, 16
NW = NC * NS
E_PER_W = N_EDGES // NW
CHUNK = 2000
N_CHUNKS = E_PER_W // CHUNK
NPAD = 10240
NPP = NPAD // PACK
STRIPE = NPAD // NS


def _xw_body(xr_ref, w_ref, o_ref):
    o_ref[...] = jnp.dot(xr_ref[...], w_ref[...])


def _tc_xw(xr, wbig):
    return pl.pallas_call(
        _xw_body,
        out_shape=jax.ShapeDtypeStruct((NP, 128), jnp.float32),
    )(xr, wbig)


def _tt_body(w_ref, ea_ref, b_ref, o_ref):
    o_ref[...] = jnp.dot(w_ref[...], ea_ref[...]) + b_ref[...]


def _tc_tt(w1t, eaT, b1c):
    grid = 2
    eb = N_EDGES // grid
    col_spec = pl.BlockSpec((N_HID, eb), lambda i: (0, i))
    return pl.pallas_call(
        _tt_body,
        grid=(grid,),
        in_specs=[pl.BlockSpec((N_HID, N_HID), lambda i: (0, 0)),
                  col_spec,
                  pl.BlockSpec((N_HID, 1), lambda i: (0, 0))],
        out_specs=col_spec,
        out_shape=jax.ShapeDtypeStruct((N_HID, N_EDGES), jnp.float32),
    )(w1t, eaT, b1c)


def _sc_gather_build():
    mesh = plsc.VectorSubcoreMesh(core_axis_name="c", subcore_axis_name="s")

    @functools.partial(
        pl.kernel,
        mesh=mesh,
        out_type=jax.ShapeDtypeStruct((N_EDGES, N_HID), jnp.float32),
        scratch_types=[
            pltpu.VMEM((2, CHUNK), jnp.int32),
            pltpu.VMEM((2, CHUNK, N_HID), jnp.float32),
            pltpu.VMEM((N_HID, CHUNK + 8), jnp.float32),
            pltpu.SemaphoreType.DMA,
            pltpu.SemaphoreType.DMA,
            pltpu.SemaphoreType.DMA,
            pltpu.SemaphoreType.DMA,
        ],
        compiler_params=pltpu.CompilerParams(use_tc_tiling_on_sc=False,
                                             needs_layout_passes=False),
    )
    def gather_k(table_hbm, eidx_hbm, tt_hbm, out_hbm,
                 idx_v, rows_v, tt_v, sg0, sg1, so0, so1):
        wid = lax.axis_index("s") * NC + lax.axis_index("c")
        lane = lax.iota(jnp.int32, 16)
        sgs = (sg0, sg1)
        sos = (so0, so1)

        def ebase(j):
            return wid * E_PER_W + j * CHUNK

        pltpu.sync_copy(eidx_hbm.at[1, pl.ds(ebase(0), CHUNK)], idx_v.at[0])
        gcp = [None, None]
        ocp = [None, None]
        gcp[0] = pltpu.async_copy(table_hbm.at[idx_v.at[0]], rows_v.at[0],
                                  sgs[0])
        for j in range(N_CHUNKS):
            b = j % 2
            nb = 1 - b
            if j + 1 < N_CHUNKS:
                pltpu.sync_copy(eidx_hbm.at[1, pl.ds(ebase(j + 1), CHUNK)],
                                idx_v.at[nb])
                if ocp[nb] is not None:
                    ocp[nb].wait()
                    ocp[nb] = None
                gcp[nb] = pltpu.async_copy(table_hbm.at[idx_v.at[nb]],
                                           rows_v.at[nb], sgs[nb])
            pltpu.sync_copy(tt_hbm.at[:, pl.ds(ebase(j), CHUNK)],
                            tt_v.at[:, pl.ds(0, CHUNK)])
            gcp[b].wait()

            def add_t(o, _):
                base_vec = jnp.full((16,), o * 16, jnp.int32)
                tc = [None] * 16
                for i in range(16):
                    tc[i] = plsc.load_gather(tt_v, [lane, base_vec + i])
                for i in range(16):
                    e = o * 16 + i
                    rows_v[b, e] = rows_v[b, e] + tc[i]
                return 0

            lax.fori_loop(0, CHUNK // 16, add_t, 0)
            ocp[b] = pltpu.async_copy(rows_v.at[b],
                                      out_hbm.at[pl.ds(ebase(j), CHUNK)],
                                      sos[b])
        for b in range(2):
            if ocp[b] is not None:
                ocp[b].wait()

    return gather_k


_gather_cache = []


def _sc_gather(table, idx, tt):
    if not _gather_cache:
        _gather_cache.append(_sc_gather_build())
    return _gather_cache[0](table, idx, tt)


def _sc_scatter_build():
    mesh = plsc.VectorSubcoreMesh(core_axis_name="c", subcore_axis_name="s")

    @functools.partial(
        pl.kernel,
        mesh=mesh,
        out_type=(
            jax.ShapeDtypeStruct((NC, NPAD, N_HID), jnp.float32),
            jax.ShapeDtypeStruct((NC, NPAD), jnp.float32),
        ),
        scratch_types=[
            pltpu.VMEM_SHARED((NPAD, N_HID), jnp.float32),
            pltpu.VMEM_SHARED((NPAD,), jnp.float32),
            pltpu.VMEM((2, CHUNK), jnp.int32),
            pltpu.VMEM((2, CHUNK, N_HID), jnp.float32),
            pltpu.VMEM((CHUNK,), jnp.float32),
            pltpu.VMEM((STRIPE, N_HID), jnp.float32),
            pltpu.VMEM((STRIPE,), jnp.float32),
            pltpu.SemaphoreType.DMA,
            pltpu.SemaphoreType.DMA,
            pltpu.SemaphoreType.DMA,
            pltpu.SemaphoreType.DMA,
        ],
        compiler_params=pltpu.CompilerParams(use_tc_tiling_on_sc=False),
    )
    def scatter_k(eidx_hbm, h_hbm, acc_out, cnt_out,
                  acc_s, cnt_s, idx_v, h_v, ones_v, zrow_v, zcnt_v,
                  si0, si1, sh0, sh1):
        c = lax.axis_index("c")
        s = lax.axis_index("s")
        sis = (si0, si1)
        shs = (sh0, sh1)

        def cbase(j):
            return c * (N_EDGES // NC) + s * E_PER_W + j * CHUNK

        def start_loads(j, bb):
            icp = pltpu.async_copy(eidx_hbm.at[0, pl.ds(cbase(j), CHUNK)],
                                   idx_v.at[bb], sis[bb])
            hcp = pltpu.async_copy(h_hbm.at[pl.ds(cbase(j), CHUNK)],
                                   h_v.at[bb], shs[bb])
            return icp, hcp

        cps = [start_loads(0, 0), None]

        def fill_rows(i, _):
            zrow_v[i] = jnp.zeros((N_HID,), jnp.float32)
            return 0

        lax.fori_loop(0, STRIPE, fill_rows, 0)

        def fill_1d(i, _):
            zcnt_v[pl.ds(i * 16, 16)] = jnp.zeros((16,), jnp.float32)
            ones_v[pl.ds(i * 16, 16)] = jnp.ones((16,), jnp.float32)
            return 0

        lax.fori_loop(0, STRIPE // 16, fill_1d, 0)

        def fill_ones_tail(i, _):
            ones_v[pl.ds(i * 16, 16)] = jnp.ones((16,), jnp.float32)
            return 0

        lax.fori_loop(STRIPE // 16, CHUNK // 16, fill_ones_tail, 0)

        pltpu.sync_copy(zrow_v, acc_s.at[pl.ds(s * STRIPE, STRIPE)])
        pltpu.sync_copy(zcnt_v, cnt_s.at[pl.ds(s * STRIPE, STRIPE)])
        plsc.subcore_barrier()

        for j in range(N_CHUNKS):
            b = j % 2
            if j + 1 < N_CHUNKS:
                cps[1 - b] = start_loads(j + 1, 1 - b)
            icp, hcp = cps[b]
            icp.wait()
            hcp.wait()
            pltpu.sync_copy(h_v.at[b], acc_s.at[idx_v.at[b]], add=True)
            pltpu.sync_copy(ones_v, cnt_s.at[idx_v.at[b]], add=True)

        plsc.subcore_barrier()
        pltpu.sync_copy(acc_s.at[pl.ds(s * STRIPE, STRIPE)],
                        acc_out.at[c, pl.ds(s * STRIPE, STRIPE)])
        pltpu.sync_copy(cnt_s.at[pl.ds(s * STRIPE, STRIPE)],
                        cnt_out.at[c, pl.ds(s * STRIPE, STRIPE)])

    return scatter_k


_scatter_cache = []


def _sc_scatter(row, h):
    if not _scatter_cache:
        _scatter_cache.append(_sc_scatter_build())
    return _scatter_cache[0](row, h)


def _edge_body(g_ref, w2c_ref, s16_ref, b2c_ref, g1_ref, be1_ref, o_ref):
    r = jnp.maximum(g_ref[...], 0.0)
    cc = jnp.dot(r, w2c_ref[...]) + b2c_ref[...]
    v = jnp.dot(cc * cc, s16_ref[...])
    o_ref[...] = cc * jax.lax.rsqrt(v + EPS) * g1_ref[...] + be1_ref[...]


def _tc_edge(g_p, w2c_blk, s16, b2c_t, g1_t, be1_t):
    grid = 2
    eb = EP // grid
    row_spec = pl.BlockSpec((eb, 128), lambda i: (i, 0))
    full = pl.BlockSpec((128, 128), lambda i: (0, 0))
    vec = pl.BlockSpec((1, 128), lambda i: (0, 0))
    return pl.pallas_call(
        _edge_body,
        grid=(grid,),
        in_specs=[row_spec, full, full, vec, vec, vec],
        out_specs=row_spec,
        out_shape=jax.ShapeDtypeStruct((EP, 128), jnp.float32),
    )(g_p, w2c_blk, s16, b2c_t, g1_t, be1_t)


def _node_body(xr_ref, acc_ref, cnt_ref, u_ref, w3x_ref, w3m_ref, w3u_ref,
               b8_ref, w4c_ref, s16_ref, b3_ref, b4c_ref, g2_ref, be2_ref,
               o_ref):
    acc = acc_ref[0][:NP] + acc_ref[1][:NP]
    cntn = cnt_ref[0][:NP] + cnt_ref[1][:NP]
    cnt_p = jnp.dot(cntn, b8_ref[...])
    mean_p = acc / jnp.maximum(cnt_p, 1.0)
    z = (jnp.dot(xr_ref[...], w3x_ref[...])
         + jnp.dot(mean_p, w3m_ref[...])
         + jnp.dot(u_ref[...], w3u_ref[...])
         + b3_ref[...])
    r = jnp.maximum(z, 0.0)
    cc = jnp.dot(r, w4c_ref[...]) + b4c_ref[...]
    v = jnp.dot(cc * cc, s16_ref[...])
    o_ref[...] = cc * jax.lax.rsqrt(v + EPS) * g2_ref[...] + be2_ref[...]


def _tc_node(xr, acc_p, cnt_r, u_p, w3x_big, w3m_blk, w3u_blk, b8, w4c_blk,
             s16, b3_t, b4c_t, g2_t, be2_t):
    return pl.pallas_call(
        _node_body,
        out_shape=jax.ShapeDtypeStruct((NP, 128), jnp.float32),
    )(xr, acc_p, cnt_r, u_p, w3x_big, w3m_blk, w3u_blk, b8, w4c_blk,
      s16, b3_t, b4c_t, g2_t, be2_t)


def kernel(x, edge_idx, edge_attr, u, W1, b1, W2, b2, g1, be1,
           W3, b3, W4, b4, g2, be2):
    eidx = edge_idx.astype(jnp.int32)

    i8 = jnp.eye(PACK, dtype=jnp.float32)
    w1x_big = jnp.kron(i8, W1[:N_FEAT])
    w1t = W1[N_FEAT:].T
    w2c = W2 - jnp.mean(W2, axis=1, keepdims=True)
    w2c_blk = jnp.kron(i8, w2c)
    w4c = W4 - jnp.mean(W4, axis=1, keepdims=True)
    w4c_blk = jnp.kron(i8, w4c)
    w3x_big = jnp.kron(i8, W3[:N_FEAT])
    w3m_blk = jnp.kron(i8, W3[N_FEAT:N_FEAT + N_HID])
    w3u_blk = jnp.kron(i8, W3[N_FEAT + N_HID:])
    s16 = jnp.kron(i8, jnp.full((N_HID, N_HID), 1.0 / N_HID, jnp.float32))
    b8 = jnp.kron(i8, jnp.ones((1, N_HID), jnp.float32))

    def tile8(v):
        return jnp.tile(v, PACK).reshape(1, 128)

    b1c = b1.reshape(N_HID, 1)
    b2c_t = tile8(b2 - jnp.mean(b2))
    b4c_t = tile8(b4 - jnp.mean(b4))
    b3_t = tile8(b3)
    g1_t, be1_t = tile8(g1), tile8(be1)
    g2_t, be2_t = tile8(g2), tile8(be2)

    xr = x.reshape(NP, PACK * N_FEAT)

    xw1 = _tc_xw(xr, w1x_big).reshape(N_NODES, N_HID)

    tt = _tc_tt(w1t, edge_attr.T, b1c)

    g = _sc_gather(xw1, eidx, tt)

    h_p = _tc_edge(g.reshape(EP, 128), w2c_blk, s16, b2c_t, g1_t, be1_t)
    h = h_p.reshape(N_EDGES, N_HID)

    acc2, cnt2 = _sc_scatter(eidx, h)

    u_late, _ = lax.optimization_barrier((u, g))

    acc_p = acc2.reshape(NC, NPP, 128)
    cnt_r = cnt2.reshape(NC, NPP, PACK)
    u_p = u_late.reshape(NP, 128)
    out_p = _tc_node(xr, acc_p, cnt_r, u_p, w3x_big, w3m_blk, w3u_blk, b8,
                     w4c_blk, s16, b3_t, b4c_t, g2_t, be2_t)
    return out_p.reshape(N_NODES, N_HID)
